# baseline (speedup 1.0000x reference)
import functools

import jax
import jax.numpy as jnp
from jax import lax
from jax.experimental import pallas as pl
from jax.experimental.pallas import tpu as pltpu
from jax.experimental.pallas import tpu_sc as plsc

R = 262144
C = 128
NUM_CORES = 2
NUM_SUBCORES = 16
BUF_ROWS = 128
LANES = 16

N_DMA_C0 = 77
N_DMA_C1 = 51
C0_ROWS = NUM_SUBCORES * N_DMA_C0 * BUF_ROWS
assert C0_ROWS + NUM_SUBCORES * N_DMA_C1 * BUF_ROWS == R


def _sc_body(sub_hbm, out_hbm, buf_v, sem):
    pltpu.sync_copy(sub_hbm, buf_v.at[0])
    vregs = [buf_v[0, pl.ds(LANES * j, LANES)] for j in range(C // LANES)]

    def fill(i, carry):
        for j in range(C // LANES):
            buf_v[i, pl.ds(LANES * j, LANES)] = vregs[j]
        return carry

    lax.fori_loop(1, BUF_ROWS, fill, 0)

    cid = lax.axis_index("c")
    sid = lax.axis_index("s")

    def emit(base, n_dma):
        copies = [
            pltpu.async_copy(
                buf_v, out_hbm.at[pl.ds(base + j * BUF_ROWS, BUF_ROWS)], sem)
            for j in range(n_dma)
        ]
        for cp in copies:
            cp.wait()

    @pl.when(cid == 0)
    def _():
        emit(sid * (N_DMA_C0 * BUF_ROWS), N_DMA_C0)

    @pl.when(cid == 1)
    def _():
        emit(C0_ROWS + sid * (N_DMA_C1 * BUF_ROWS), N_DMA_C1)


_sc_broadcast = functools.partial(
    pl.kernel,
    mesh=plsc.VectorSubcoreMesh(core_axis_name="c", subcore_axis_name="s"),
    out_type=jax.ShapeDtypeStruct((R, C), jnp.float32),
    scratch_types=[
        pltpu.VMEM((BUF_ROWS, C), jnp.float32),
        pltpu.SemaphoreType.DMA,
    ],
)(_sc_body)


def kernel(input_2d_tensor, substitution_tensor):
    del input_2d_tensor
    return _sc_broadcast(substitution_tensor)


# speedup vs baseline: 1.0774x; 1.0774x over previous
import functools

import jax
import jax.numpy as jnp
from jax import lax
from jax.experimental import pallas as pl
from jax.experimental.pallas import tpu as pltpu
from jax.experimental.pallas import tpu_sc as plsc

R = 262144
C = 128
NUM_CORES = 2
NUM_SUBCORES = 16
NUM_WORKERS = NUM_CORES * NUM_SUBCORES
ROWS_PER_WORKER = R // NUM_WORKERS
BUF_ROWS = 128
N_DMA = ROWS_PER_WORKER // BUF_ROWS
LANES = 16


def _sc_body(sub_hbm, out_hbm, buf_v, shared_v, sem):
    pltpu.sync_copy(sub_hbm, buf_v.at[0])
    vregs = [buf_v[0, pl.ds(LANES * j, LANES)] for j in range(C // LANES)]

    def fill(i, carry):
        for j in range(C // LANES):
            buf_v[i, pl.ds(LANES * j, LANES)] = vregs[j]
        return carry

    lax.fori_loop(1, BUF_ROWS, fill, 0)

    cid = lax.axis_index("c")
    sid = lax.axis_index("s")

    @pl.when(sid == 0)
    def _():
        pltpu.sync_copy(buf_v, shared_v)

    plsc.subcore_barrier()

    base = (cid * NUM_SUBCORES + sid) * ROWS_PER_WORKER
    copies = [
        pltpu.async_copy(
            buf_v if j % 2 == 0 else shared_v,
            out_hbm.at[pl.ds(base + j * BUF_ROWS, BUF_ROWS)], sem)
        for j in range(N_DMA)
    ]
    for cp in copies:
        cp.wait()


_sc_broadcast = functools.partial(
    pl.kernel,
    mesh=plsc.VectorSubcoreMesh(core_axis_name="c", subcore_axis_name="s"),
    out_type=jax.ShapeDtypeStruct((R, C), jnp.float32),
    scratch_types=[
        pltpu.VMEM((BUF_ROWS, C), jnp.float32),
        pltpu.VMEM_SHARED((BUF_ROWS, C), jnp.float32),
        pltpu.SemaphoreType.DMA,
    ],
)(_sc_body)


def kernel(input_2d_tensor, substitution_tensor):
    del input_2d_tensor
    return _sc_broadcast(substitution_tensor)


# speedup vs baseline: 1.1056x; 1.0261x over previous
import functools

import jax
import jax.numpy as jnp
from jax import lax
from jax.experimental import pallas as pl
from jax.experimental.pallas import tpu as pltpu
from jax.experimental.pallas import tpu_sc as plsc

R = 262144
C = 128
NUM_CORES = 2
NUM_SUBCORES = 16
NUM_WORKERS = NUM_CORES * NUM_SUBCORES
ROWS_PER_WORKER = R // NUM_WORKERS
BUF_ROWS = 128
N_DMA = ROWS_PER_WORKER // BUF_ROWS
LANES = 16


def _sc_body(sub_hbm, out_hbm, buf_v, sem):
    pltpu.sync_copy(sub_hbm, buf_v.at[0])
    vregs = [buf_v[0, pl.ds(LANES * j, LANES)] for j in range(C // LANES)]

    def fill(i, carry):
        for j in range(C // LANES):
            buf_v[i, pl.ds(LANES * j, LANES)] = vregs[j]
        return carry

    lax.fori_loop(1, BUF_ROWS, fill, 0)

    wid = lax.axis_index("c") * NUM_SUBCORES + lax.axis_index("s")
    base = wid * ROWS_PER_WORKER
    copies = [
        pltpu.async_copy(
            buf_v, out_hbm.at[pl.ds(base + j * BUF_ROWS, BUF_ROWS)], sem)
        for j in range(N_DMA)
    ]
    for cp in copies:
        cp.wait()


_sc_broadcast = functools.partial(
    pl.kernel,
    mesh=plsc.VectorSubcoreMesh(core_axis_name="c", subcore_axis_name="s"),
    out_type=jax.ShapeDtypeStruct((R, C), jnp.float32),
    scratch_types=[
        pltpu.VMEM((BUF_ROWS, C), jnp.float32),
        pltpu.SemaphoreType.DMA,
    ],
)(_sc_body)


def kernel(input_2d_tensor, substitution_tensor):
    del input_2d_tensor
    return _sc_broadcast(substitution_tensor)
